# baseline (speedup 1.0000x reference)
import functools

import jax
import jax.numpy as jnp
from jax import lax
from jax.experimental import pallas as pl
from jax.experimental.pallas import tpu as pltpu
from jax.experimental.pallas import tpu_sc as plsc

N = 10000
DI = 128
DO = 128
E_RAW = 320000
E_TOT = E_RAW + N
NC = 2
NS = 16
NW = NC * NS
CH = 128
NBUF = 2
NPH = 2
PHC = 42
NCHUNK = NPH * PHC
E_PAD = NW * NCHUNK * CH
NPAD = 10240
RPT = NPAD // NS
NPZ = 10240
RPTZ = NPZ // NS

GRID_SIZE = 5
SPLINE_ORDER = 3
NB = GRID_SIZE + SPLINE_ORDER
H = 2.0 / GRID_SIZE
T0 = -1.0 - SPLINE_ORDER * H

@functools.cache
def _mesh():
    return plsc.VectorSubcoreMesh(
        core_axis_name="c", subcore_axis_name="s", num_cores=NC, num_subcores=NS)


def _sc_hist_body(dst_hbm, zdeg_hbm, deg_out, didx, ones, degsh, sem):
    c = lax.axis_index("c")
    s = lax.axis_index("s")
    wid = s * NC + c
    for i in range(CH // 16):
        ones[pl.ds(i * 16, 16)] = jnp.ones((16,), jnp.float32)
    pltpu.sync_copy(zdeg_hbm.at[pl.ds(s * RPT, RPT)], degsh.at[pl.ds(s * RPT, RPT)])
    pltpu.async_copy(dst_hbm.at[wid], didx, sem).wait()
    plsc.subcore_barrier()

    def step(j, carry):
        pltpu.sync_copy(ones, degsh.at[didx.at[j]], add=True)
        return carry

    lax.fori_loop(0, NCHUNK, step, 0)
    plsc.subcore_barrier()
    pltpu.sync_copy(degsh.at[pl.ds(s * RPT, RPT)], deg_out.at[c, pl.ds(s * RPT, RPT)])


@functools.cache
def _sc_hist():
    return pl.kernel(
        _sc_hist_body,
        out_type=jax.ShapeDtypeStruct((NC, NPAD), jnp.float32),
        mesh=_mesh(),
        scratch_types=[
            pltpu.VMEM((NCHUNK, CH), jnp.int32),
            pltpu.VMEM((CH,), jnp.float32),
            pltpu.VMEM_SHARED((NPAD,), jnp.float32),
            pltpu.SemaphoreType.DMA,
        ],
    )


def _tc_scale_body(x_ref, deg_ref, y_ref):
    deg = deg_ref[0] + deg_ref[1]
    dinv = lax.rsqrt(jnp.maximum(deg, 1.0))
    y_ref[...] = x_ref[...] * dinv


def _tc_scale(x, deg2):
    R = 400
    grid = N // R
    deg3 = deg2.reshape(NC, NPAD, 1)
    return pl.pallas_call(
        _tc_scale_body,
        grid=(grid,),
        in_specs=[
            pl.BlockSpec((R, DI), lambda i: (i, 0)),
            pl.BlockSpec((NC, R, 1), lambda i: (0, i, 0)),
        ],
        out_specs=pl.BlockSpec((R, DI), lambda i: (i, 0)),
        out_shape=jax.ShapeDtypeStruct((N, DI), jnp.float32),
    )(x, deg3)


def _sc_agg_body(src_hbm, dst_hbm, y_hbm, zrows_hbm, z_out,
                 sidx, didx, rows0, rows1, zsh, isem, g0, g1):
    c = lax.axis_index("c")
    s = lax.axis_index("s")
    wid = s * NC + c
    rows = (rows0, rows1)
    gsems = (g0, g1)
    pltpu.sync_copy(zrows_hbm.at[pl.ds(s * RPTZ, RPTZ)],
                    zsh.at[pl.ds(s * RPTZ, RPTZ)])

    def stage_idx(p):
        pltpu.async_copy(src_hbm.at[wid, p], sidx, isem)
        pltpu.async_copy(dst_hbm.at[wid, p], didx, isem)

    def wait_idx(p):
        pltpu.make_async_copy(src_hbm.at[wid, p], sidx, isem).wait()
        pltpu.make_async_copy(dst_hbm.at[wid, p], didx, isem).wait()

    def scatter(local, b):
        pltpu.sync_copy(rows[b], zsh.at[didx.at[local]], add=True)

    stage_idx(0)
    wait_idx(0)
    plsc.subcore_barrier()

    for p in range(NPH):
        if p > 0:
            wait_idx(p)

        def step(i, carry):
            cp0 = pltpu.async_copy(y_hbm.at[sidx.at[i]], rows0, g0)
            cp0.wait()
            scatter(i, 0)
            return carry

        lax.fori_loop(0, PHC, step, 0)
        if p + 1 < NPH:
            stage_idx(p + 1)

    plsc.subcore_barrier()
    pltpu.sync_copy(zsh.at[pl.ds(s * RPTZ, RPTZ)],
                    z_out.at[c, pl.ds(s * RPTZ, RPTZ)])


@functools.cache
def _sc_agg():
    return pl.kernel(
        _sc_agg_body,
        out_type=jax.ShapeDtypeStruct((NC, NPZ, DI), jnp.float32),
        mesh=_mesh(),
        scratch_types=[
            pltpu.VMEM((PHC, CH), jnp.int32),
            pltpu.VMEM((PHC, CH), jnp.int32),
            pltpu.VMEM((CH, DI), jnp.float32),
            pltpu.VMEM((CH, DI), jnp.float32),
            pltpu.VMEM_SHARED((NPZ, DI), jnp.float32),
            pltpu.SemaphoreType.DMA,
            pltpu.SemaphoreType.DMA,
            pltpu.SemaphoreType.DMA,
        ],
    )


def _bspline_bases(v):
    u = (v - T0) * (1.0 / H)
    m = jnp.floor(u)
    f = u - m
    mi = m.astype(jnp.int32)
    f2 = f * f
    f3 = f2 * f
    one_sixth = 1.0 / 6.0
    p0 = f3 * one_sixth
    p1 = (-3.0 * f3 + 3.0 * f2 + 3.0 * f + 1.0) * one_sixth
    p2 = (3.0 * f3 - 6.0 * f2 + 4.0) * one_sixth
    g = 1.0 - f
    p3 = g * g * g * one_sixth
    zero = jnp.zeros_like(v)
    bases = []
    for j in range(NB):
        b = jnp.where(mi == j, p0, zero)
        b = jnp.where(mi == j + 1, p1, b)
        b = jnp.where(mi == j + 2, p2, b)
        b = jnp.where(mi == j + 3, p3, b)
        bases.append(b)
    return bases


def _kan(v, wb_ref, ws_ref):
    base = jnp.dot(v * jax.nn.sigmoid(v), wb_ref[...],
                   preferred_element_type=jnp.float32)
    acc = base
    for j, bj in enumerate(_bspline_bases(v)):
        acc = acc + jnp.dot(bj, ws_ref[j], preferred_element_type=jnp.float32)
    return acc


def _tc_kan_body(x_ref, z_ref, deg_ref, wbl_ref, wsl_ref, wbc_ref, wsc_ref, out_ref):
    xb = x_ref[...]
    zb = z_ref[0] + z_ref[1]
    deg = deg_ref[0] + deg_ref[1]
    dinv = lax.rsqrt(jnp.maximum(deg, 1.0))
    agg = zb * dinv
    out_ref[...] = _kan(xb, wbl_ref, wsl_ref) + _kan(agg, wbc_ref, wsc_ref)


def _tc_kan(x, zfull, deg2, wbl_t, wsl_t, wbc_t, wsc_t):
    R = 400
    grid = N // R
    deg3 = deg2.reshape(NC, NPAD, 1)
    wspec = pl.BlockSpec((DI, DO), lambda i: (0, 0))
    sspec = pl.BlockSpec((NB, DI, DO), lambda i: (0, 0, 0))
    return pl.pallas_call(
        _tc_kan_body,
        grid=(grid,),
        in_specs=[
            pl.BlockSpec((R, DI), lambda i: (i, 0)),
            pl.BlockSpec((NC, R, DI), lambda i: (0, i, 0)),
            pl.BlockSpec((NC, R, 1), lambda i: (0, i, 0)),
            wspec, sspec, wspec, sspec,
        ],
        out_specs=pl.BlockSpec((R, DO), lambda i: (i, 0)),
        out_shape=jax.ShapeDtypeStruct((N, DO), jnp.float32),
    )(x, zfull, deg3, wbl_t, wsl_t, wbc_t, wsc_t)


def kernel(x, base_w_l, spline_w_l, base_w_c, spline_w_c, edge_index):
    loop = jnp.arange(N, dtype=jnp.int32)
    pad_src = jnp.zeros((E_PAD - E_TOT,), jnp.int32)
    pad_dst = jnp.full((E_PAD - E_TOT,), N, jnp.int32)
    src_flat = jnp.concatenate([edge_index[0], loop, pad_src])
    dst_flat = jnp.concatenate([edge_index[1], loop, pad_dst])
    src4 = src_flat.reshape(NW, NPH, PHC, CH)
    dst4 = dst_flat.reshape(NW, NPH, PHC, CH)
    dst3 = dst_flat.reshape(NW, NCHUNK, CH)
    zdeg = jnp.zeros((NPAD,), jnp.float32)
    zrows = jnp.zeros((NPZ, DI), jnp.float32)
    wbl_t = base_w_l.T
    wbc_t = base_w_c.T
    wsl_t = jnp.transpose(spline_w_l, (2, 1, 0))
    wsc_t = jnp.transpose(spline_w_c, (2, 1, 0))

    deg2 = _sc_hist()(dst3, zdeg)
    y = _tc_scale(x, deg2)
    zfull = _sc_agg()(src4, dst4, y, zrows)
    return _tc_kan(x, zfull, deg2, wbl_t, wsl_t, wbc_t, wsc_t)


# speedup vs baseline: 1.0251x; 1.0251x over previous
import functools

import jax
import jax.numpy as jnp
from jax import lax
from jax.experimental import pallas as pl
from jax.experimental.pallas import tpu as pltpu
from jax.experimental.pallas import tpu_sc as plsc

N = 10000
DI = 128
DO = 128
E_RAW = 320000
E_TOT = E_RAW + N
NC = 2
NS = 16
NW = NC * NS
CH = 128
NBUF = 2
NPH = 2
PHC = 42
NCHUNK = NPH * PHC
E_PAD = NW * NCHUNK * CH
NPAD = 10240
RPT = NPAD // NS
NPZ = 10240
RPTZ = NPZ // NS

GRID_SIZE = 5
SPLINE_ORDER = 3
NB = GRID_SIZE + SPLINE_ORDER
H = 2.0 / GRID_SIZE
T0 = -1.0 - SPLINE_ORDER * H

@functools.cache
def _mesh():
    return plsc.VectorSubcoreMesh(
        core_axis_name="c", subcore_axis_name="s", num_cores=NC, num_subcores=NS)


def _sc_hist_body(dst_hbm, zdeg_hbm, deg_out, didx, ones, degsh, sem):
    c = lax.axis_index("c")
    s = lax.axis_index("s")
    wid = s * NC + c
    for i in range(CH // 16):
        ones[pl.ds(i * 16, 16)] = jnp.ones((16,), jnp.float32)
    pltpu.sync_copy(zdeg_hbm.at[pl.ds(s * RPT, RPT)], degsh.at[pl.ds(s * RPT, RPT)])
    pltpu.async_copy(dst_hbm.at[wid], didx, sem).wait()
    plsc.subcore_barrier()

    def step(j, carry):
        pltpu.sync_copy(ones, degsh.at[didx.at[j]], add=True)
        return carry

    lax.fori_loop(0, NCHUNK, step, 0)
    plsc.subcore_barrier()
    pltpu.sync_copy(degsh.at[pl.ds(s * RPT, RPT)], deg_out.at[c, pl.ds(s * RPT, RPT)])


@functools.cache
def _sc_hist():
    return pl.kernel(
        _sc_hist_body,
        out_type=jax.ShapeDtypeStruct((NC, NPAD), jnp.float32),
        mesh=_mesh(),
        scratch_types=[
            pltpu.VMEM((NCHUNK, CH), jnp.int32),
            pltpu.VMEM((CH,), jnp.float32),
            pltpu.VMEM_SHARED((NPAD,), jnp.float32),
            pltpu.SemaphoreType.DMA,
        ],
    )


def _tc_scale_body(x_ref, deg_ref, y_ref):
    deg = deg_ref[0] + deg_ref[1]
    dinv = lax.rsqrt(jnp.maximum(deg, 1.0))
    y_ref[...] = x_ref[...] * dinv


def _tc_scale(x, deg2):
    R = 400
    grid = N // R
    deg3 = deg2.reshape(NC, NPAD, 1)
    return pl.pallas_call(
        _tc_scale_body,
        grid=(grid,),
        in_specs=[
            pl.BlockSpec((R, DI), lambda i: (i, 0)),
            pl.BlockSpec((NC, R, 1), lambda i: (0, i, 0)),
        ],
        out_specs=pl.BlockSpec((R, DI), lambda i: (i, 0)),
        out_shape=jax.ShapeDtypeStruct((N, DI), jnp.float32),
    )(x, deg3)


def _sc_agg_body(src_hbm, dst_hbm, y_hbm, zrows_hbm, z_out,
                 sidx, didx, rows0, rows1, zsh, isem, g0, g1):
    c = lax.axis_index("c")
    s = lax.axis_index("s")
    wid = s * NC + c
    rows = (rows0, rows1)
    gsems = (g0, g1)
    pltpu.sync_copy(zrows_hbm.at[pl.ds(s * RPTZ, RPTZ)],
                    zsh.at[pl.ds(s * RPTZ, RPTZ)])

    def stage_idx(p):
        pltpu.async_copy(src_hbm.at[wid, p], sidx, isem)
        pltpu.async_copy(dst_hbm.at[wid, p], didx, isem)

    def wait_idx(p):
        pltpu.make_async_copy(src_hbm.at[wid, p], sidx, isem).wait()
        pltpu.make_async_copy(dst_hbm.at[wid, p], didx, isem).wait()

    def scatter(local, b):
        pltpu.sync_copy(rows[b], zsh.at[didx.at[local]], add=True)

    stage_idx(0)
    wait_idx(0)
    plsc.subcore_barrier()

    for p in range(NPH):
        if p > 0:
            wait_idx(p)

        def pair(i, carry):
            cp0 = pltpu.async_copy(y_hbm.at[sidx.at[2 * i]], rows0, g0)
            cp1 = pltpu.async_copy(y_hbm.at[sidx.at[2 * i + 1]], rows1, g1)
            cp0.wait()
            scatter(2 * i, 0)
            cp1.wait()
            scatter(2 * i + 1, 1)
            return carry

        lax.fori_loop(0, PHC // 2, pair, 0)
        if p + 1 < NPH:
            stage_idx(p + 1)

    plsc.subcore_barrier()
    pltpu.sync_copy(zsh.at[pl.ds(s * RPTZ, RPTZ)],
                    z_out.at[c, pl.ds(s * RPTZ, RPTZ)])


@functools.cache
def _sc_agg():
    return pl.kernel(
        _sc_agg_body,
        out_type=jax.ShapeDtypeStruct((NC, NPZ, DI), jnp.float32),
        mesh=_mesh(),
        scratch_types=[
            pltpu.VMEM((PHC, CH), jnp.int32),
            pltpu.VMEM((PHC, CH), jnp.int32),
            pltpu.VMEM((CH, DI), jnp.float32),
            pltpu.VMEM((CH, DI), jnp.float32),
            pltpu.VMEM_SHARED((NPZ, DI), jnp.float32),
            pltpu.SemaphoreType.DMA,
            pltpu.SemaphoreType.DMA,
            pltpu.SemaphoreType.DMA,
        ],
    )


def _bspline_bases(v):
    u = (v - T0) * (1.0 / H)
    m = jnp.floor(u)
    f = u - m
    mi = m.astype(jnp.int32)
    f2 = f * f
    f3 = f2 * f
    one_sixth = 1.0 / 6.0
    p0 = f3 * one_sixth
    p1 = (-3.0 * f3 + 3.0 * f2 + 3.0 * f + 1.0) * one_sixth
    p2 = (3.0 * f3 - 6.0 * f2 + 4.0) * one_sixth
    g = 1.0 - f
    p3 = g * g * g * one_sixth
    zero = jnp.zeros_like(v)
    bases = []
    for j in range(NB):
        b = jnp.where(mi == j, p0, zero)
        b = jnp.where(mi == j + 1, p1, b)
        b = jnp.where(mi == j + 2, p2, b)
        b = jnp.where(mi == j + 3, p3, b)
        bases.append(b)
    return bases


def _kan(v, wb_ref, ws_ref):
    base = jnp.dot(v * jax.nn.sigmoid(v), wb_ref[...],
                   preferred_element_type=jnp.float32)
    acc = base
    for j, bj in enumerate(_bspline_bases(v)):
        acc = acc + jnp.dot(bj, ws_ref[j], preferred_element_type=jnp.float32)
    return acc


def _tc_kan_body(x_ref, z_ref, deg_ref, wbl_ref, wsl_ref, wbc_ref, wsc_ref, out_ref):
    xb = x_ref[...]
    zb = z_ref[0] + z_ref[1]
    deg = deg_ref[0] + deg_ref[1]
    dinv = lax.rsqrt(jnp.maximum(deg, 1.0))
    agg = zb * dinv
    out_ref[...] = _kan(xb, wbl_ref, wsl_ref) + _kan(agg, wbc_ref, wsc_ref)


def _tc_kan(x, zfull, deg2, wbl_t, wsl_t, wbc_t, wsc_t):
    R = 400
    grid = N // R
    deg3 = deg2.reshape(NC, NPAD, 1)
    wspec = pl.BlockSpec((DI, DO), lambda i: (0, 0))
    sspec = pl.BlockSpec((NB, DI, DO), lambda i: (0, 0, 0))
    return pl.pallas_call(
        _tc_kan_body,
        grid=(grid,),
        in_specs=[
            pl.BlockSpec((R, DI), lambda i: (i, 0)),
            pl.BlockSpec((NC, R, DI), lambda i: (0, i, 0)),
            pl.BlockSpec((NC, R, 1), lambda i: (0, i, 0)),
            wspec, sspec, wspec, sspec,
        ],
        out_specs=pl.BlockSpec((R, DO), lambda i: (i, 0)),
        out_shape=jax.ShapeDtypeStruct((N, DO), jnp.float32),
    )(x, zfull, deg3, wbl_t, wsl_t, wbc_t, wsc_t)


def kernel(x, base_w_l, spline_w_l, base_w_c, spline_w_c, edge_index):
    loop = jnp.arange(N, dtype=jnp.int32)
    pad_src = jnp.zeros((E_PAD - E_TOT,), jnp.int32)
    pad_dst = N + jnp.arange(E_PAD - E_TOT, dtype=jnp.int32) % (NPZ - N)
    src_flat = jnp.concatenate([edge_index[0], loop, pad_src])
    dst_flat = jnp.concatenate([edge_index[1], loop, pad_dst])
    src4 = src_flat.reshape(NW, NPH, PHC, CH)
    dst4 = dst_flat.reshape(NW, NPH, PHC, CH)
    dst3 = dst_flat.reshape(NW, NCHUNK, CH)
    zdeg = jnp.zeros((NPAD,), jnp.float32)
    zrows = jnp.zeros((NPZ, DI), jnp.float32)
    wbl_t = base_w_l.T
    wbc_t = base_w_c.T
    wsl_t = jnp.transpose(spline_w_l, (2, 1, 0))
    wsc_t = jnp.transpose(spline_w_c, (2, 1, 0))

    deg2 = _sc_hist()(dst3, zdeg)
    y = _tc_scale(x, deg2)
    zfull = _sc_agg()(src4, dst4, y, zrows)
    return _tc_kan(x, zfull, deg2, wbl_t, wsl_t, wbc_t, wsc_t)


# speedup vs baseline: 3.2038x; 3.1253x over previous
import functools

import jax
import jax.numpy as jnp
from jax import lax
from jax.experimental import pallas as pl
from jax.experimental.pallas import tpu as pltpu
from jax.experimental.pallas import tpu_sc as plsc

N = 10000
DI = 128
DO = 128
E_RAW = 320000
E_TOT = E_RAW + N
NC = 2
NS = 16
NW = NC * NS
CH = 128
NBUF = 2
NPH = 2
PHC = 42
NCHUNK = NPH * PHC
E_PAD = NW * NCHUNK * CH
NPAD = 10240
RPT = NPAD // NS
NPZ = 10240
RPTZ = NPZ // NS

GRID_SIZE = 5
SPLINE_ORDER = 3
NB = GRID_SIZE + SPLINE_ORDER
H = 2.0 / GRID_SIZE
T0 = -1.0 - SPLINE_ORDER * H

@functools.cache
def _mesh():
    return plsc.VectorSubcoreMesh(
        core_axis_name="c", subcore_axis_name="s", num_cores=NC, num_subcores=NS)


def _sc_hist_body(dst_hbm, zdeg_hbm, deg_out, didx, ones, degsh, sem):
    c = lax.axis_index("c")
    s = lax.axis_index("s")
    wid = s * NC + c
    for i in range(CH // 16):
        ones[pl.ds(i * 16, 16)] = jnp.ones((16,), jnp.float32)
    pltpu.sync_copy(zdeg_hbm.at[pl.ds(s * RPT, RPT)], degsh.at[pl.ds(s * RPT, RPT)])
    pltpu.async_copy(dst_hbm.at[wid], didx, sem).wait()
    plsc.subcore_barrier()

    def step(j, carry):
        pltpu.sync_copy(ones, degsh.at[didx.at[j]], add=True)
        return carry

    lax.fori_loop(0, NCHUNK, step, 0)
    plsc.subcore_barrier()
    pltpu.sync_copy(degsh.at[pl.ds(s * RPT, RPT)], deg_out.at[c, pl.ds(s * RPT, RPT)])


@functools.cache
def _sc_hist():
    return pl.kernel(
        _sc_hist_body,
        out_type=jax.ShapeDtypeStruct((NC, NPAD), jnp.float32),
        mesh=_mesh(),
        scratch_types=[
            pltpu.VMEM((NCHUNK, CH), jnp.int32),
            pltpu.VMEM((CH,), jnp.float32),
            pltpu.VMEM_SHARED((NPAD,), jnp.float32),
            pltpu.SemaphoreType.DMA,
        ],
    )


def _tc_scale_body(x_ref, deg_ref, y_ref):
    deg = deg_ref[0] + deg_ref[1]
    dinv = lax.rsqrt(jnp.maximum(deg, 1.0))
    y_ref[...] = x_ref[...] * dinv


def _tc_scale(x, deg2):
    R = 400
    grid = N // R
    deg3 = deg2.reshape(NC, NPAD, 1)
    return pl.pallas_call(
        _tc_scale_body,
        grid=(grid,),
        in_specs=[
            pl.BlockSpec((R, DI), lambda i: (i, 0)),
            pl.BlockSpec((NC, R, 1), lambda i: (0, i, 0)),
        ],
        out_specs=pl.BlockSpec((R, DI), lambda i: (i, 0)),
        out_shape=jax.ShapeDtypeStruct((N, DI), jnp.float32),
    )(x, deg3)


def _sc_agg_body(src_hbm, dst_hbm, y_hbm, zrows_hbm, z_out,
                 sidx, didx, rows0, rows1, zsh, isem, g0, g1):
    c = lax.axis_index("c")
    s = lax.axis_index("s")
    wid = s * NC + c
    rows = (rows0, rows1)
    gsems = (g0, g1)
    pltpu.sync_copy(zrows_hbm.at[pl.ds(s * RPTZ, RPTZ)],
                    zsh.at[pl.ds(s * RPTZ, RPTZ)])

    def stage_idx(p):
        pltpu.async_copy(src_hbm.at[wid, p], sidx, isem)
        pltpu.async_copy(dst_hbm.at[wid, p], didx, isem)

    def wait_idx(p):
        pltpu.make_async_copy(src_hbm.at[wid, p], sidx, isem).wait()
        pltpu.make_async_copy(dst_hbm.at[wid, p], didx, isem).wait()

    def scatter(local, b):
        pltpu.sync_copy(rows[b], zsh.at[didx.at[local]], add=True)

    stage_idx(0)
    wait_idx(0)
    plsc.subcore_barrier()

    for p in range(NPH):
        if p > 0:
            wait_idx(p)

        def pair(i, carry):
            cp0 = pltpu.async_copy(y_hbm.at[sidx.at[2 * i]], rows0, g0)
            cp1 = pltpu.async_copy(y_hbm.at[sidx.at[2 * i + 1]], rows1, g1)
            cp0.wait()
            scatter(2 * i, 0)
            cp1.wait()
            scatter(2 * i + 1, 1)
            return carry

        lax.fori_loop(0, PHC // 2, pair, 0)
        if p + 1 < NPH:
            stage_idx(p + 1)

    plsc.subcore_barrier()
    pltpu.sync_copy(zsh.at[pl.ds(s * RPTZ, RPTZ)],
                    z_out.at[c, pl.ds(s * RPTZ, RPTZ)])


@functools.cache
def _sc_agg():
    return pl.kernel(
        _sc_agg_body,
        out_type=jax.ShapeDtypeStruct((NC, NPZ, DI), jnp.float32),
        mesh=_mesh(),
        scratch_types=[
            pltpu.VMEM((PHC, CH), jnp.int32),
            pltpu.VMEM((PHC, CH), jnp.int32),
            pltpu.VMEM((CH, DI), jnp.float32),
            pltpu.VMEM((CH, DI), jnp.float32),
            pltpu.VMEM_SHARED((NPZ, DI), jnp.float32),
            pltpu.SemaphoreType.DMA,
            pltpu.SemaphoreType.DMA,
            pltpu.SemaphoreType.DMA,
        ],
    )


def _bspline_bases(v):
    u = (v - T0) * (1.0 / H)
    m = jnp.floor(u)
    f = u - m
    mi = m.astype(jnp.int32)
    f2 = f * f
    f3 = f2 * f
    one_sixth = 1.0 / 6.0
    p0 = f3 * one_sixth
    p1 = (-3.0 * f3 + 3.0 * f2 + 3.0 * f + 1.0) * one_sixth
    p2 = (3.0 * f3 - 6.0 * f2 + 4.0) * one_sixth
    g = 1.0 - f
    p3 = g * g * g * one_sixth
    zero = jnp.zeros_like(v)
    bases = []
    for j in range(NB):
        b = jnp.where(mi == j, p0, zero)
        b = jnp.where(mi == j + 1, p1, b)
        b = jnp.where(mi == j + 2, p2, b)
        b = jnp.where(mi == j + 3, p3, b)
        bases.append(b)
    return bases


def _kan(v, wb_ref, ws_ref):
    base = jnp.dot(v * jax.nn.sigmoid(v), wb_ref[...],
                   preferred_element_type=jnp.float32)
    acc = base
    for j, bj in enumerate(_bspline_bases(v)):
        acc = acc + jnp.dot(bj, ws_ref[j], preferred_element_type=jnp.float32)
    return acc


def _tc_kan_body(x_ref, z_ref, deg_ref, wbl_ref, wsl_ref, wbc_ref, wsc_ref, out_ref):
    xb = x_ref[...]
    zb = z_ref[0] + z_ref[1]
    deg = deg_ref[0] + deg_ref[1]
    dinv = lax.rsqrt(jnp.maximum(deg, 1.0))
    agg = zb * dinv
    out_ref[...] = _kan(xb, wbl_ref, wsl_ref) + _kan(agg, wbc_ref, wsc_ref)


def _tc_kan(x, zfull, deg2, wbl_t, wsl_t, wbc_t, wsc_t):
    R = 400
    grid = N // R
    deg3 = deg2.reshape(NC, NPAD, 1)
    wspec = pl.BlockSpec((DI, DO), lambda i: (0, 0))
    sspec = pl.BlockSpec((NB, DI, DO), lambda i: (0, 0, 0))
    return pl.pallas_call(
        _tc_kan_body,
        grid=(grid,),
        in_specs=[
            pl.BlockSpec((R, DI), lambda i: (i, 0)),
            pl.BlockSpec((NC, R, DI), lambda i: (0, i, 0)),
            pl.BlockSpec((NC, R, 1), lambda i: (0, i, 0)),
            wspec, sspec, wspec, sspec,
        ],
        out_specs=pl.BlockSpec((R, DO), lambda i: (i, 0)),
        out_shape=jax.ShapeDtypeStruct((N, DO), jnp.float32),
    )(x, zfull, deg3, wbl_t, wsl_t, wbc_t, wsc_t)


def kernel(x, base_w_l, spline_w_l, base_w_c, spline_w_c, edge_index):
    loop = jnp.arange(N, dtype=jnp.int32)
    pad_src = jnp.arange(E_PAD - E_TOT, dtype=jnp.int32) % N
    pad_dst = N + jnp.arange(E_PAD - E_TOT, dtype=jnp.int32) % (NPZ - N)
    src_flat = jnp.concatenate([edge_index[0], loop, pad_src])
    dst_flat = jnp.concatenate([edge_index[1], loop, pad_dst])
    src4 = src_flat.reshape(NW, NPH, PHC, CH)
    dst4 = dst_flat.reshape(NW, NPH, PHC, CH)
    dst3 = dst_flat.reshape(NW, NCHUNK, CH)
    zdeg = jnp.zeros((NPAD,), jnp.float32)
    zrows = jnp.zeros((NPZ, DI), jnp.float32)
    wbl_t = base_w_l.T
    wbc_t = base_w_c.T
    wsl_t = jnp.transpose(spline_w_l, (2, 1, 0))
    wsc_t = jnp.transpose(spline_w_c, (2, 1, 0))

    deg2 = _sc_hist()(dst3, zdeg)
    y = _tc_scale(x, deg2)
    zfull = _sc_agg()(src4, dst4, y, zrows)
    return _tc_kan(x, zfull, deg2, wbl_t, wsl_t, wbc_t, wsc_t)


# speedup vs baseline: 3.8238x; 1.1935x over previous
import functools

import jax
import jax.numpy as jnp
from jax import lax
from jax.experimental import pallas as pl
from jax.experimental.pallas import tpu as pltpu
from jax.experimental.pallas import tpu_sc as plsc

N = 10000
DI = 128
DO = 128
E_RAW = 320000
E_TOT = E_RAW + N
NC = 2
NS = 16
NW = NC * NS
CH = 128
NBUF = 2
NPH = 2
PHC = 42
NCHUNK = NPH * PHC
E_PAD = NW * NCHUNK * CH
NPAD = 10240
RPT = NPAD // NS
NPZ = 10240
RPTZ = NPZ // NS

GRID_SIZE = 5
SPLINE_ORDER = 3
NB = GRID_SIZE + SPLINE_ORDER
H = 2.0 / GRID_SIZE
T0 = -1.0 - SPLINE_ORDER * H

@functools.cache
def _mesh():
    return plsc.VectorSubcoreMesh(
        core_axis_name="c", subcore_axis_name="s", num_cores=NC, num_subcores=NS)


def _sc_hist_body(dst_hbm, zdeg_hbm, deg_out, didx, ones, degsh, sem):
    c = lax.axis_index("c")
    s = lax.axis_index("s")
    wid = s * NC + c
    for i in range(CH // 16):
        ones[pl.ds(i * 16, 16)] = jnp.ones((16,), jnp.float32)
    pltpu.sync_copy(zdeg_hbm.at[pl.ds(s * RPT, RPT)], degsh.at[pl.ds(s * RPT, RPT)])
    pltpu.async_copy(dst_hbm.at[wid], didx, sem).wait()
    plsc.subcore_barrier()

    def step(j, carry):
        pltpu.sync_copy(ones, degsh.at[didx.at[j]], add=True)
        return carry

    lax.fori_loop(0, NCHUNK, step, 0)
    plsc.subcore_barrier()
    pltpu.sync_copy(degsh.at[pl.ds(s * RPT, RPT)], deg_out.at[c, pl.ds(s * RPT, RPT)])


@functools.cache
def _sc_hist():
    return pl.kernel(
        _sc_hist_body,
        out_type=jax.ShapeDtypeStruct((NC, NPAD), jnp.float32),
        mesh=_mesh(),
        scratch_types=[
            pltpu.VMEM((NCHUNK, CH), jnp.int32),
            pltpu.VMEM((CH,), jnp.float32),
            pltpu.VMEM_SHARED((NPAD,), jnp.float32),
            pltpu.SemaphoreType.DMA,
        ],
    )


def _tc_scale_body(x_ref, deg_ref, y_ref):
    deg = deg_ref[0] + deg_ref[1]
    dinv = lax.rsqrt(jnp.maximum(deg, 1.0))
    y_ref[...] = x_ref[...] * dinv


def _tc_scale(x, deg2):
    R = 400
    grid = N // R
    deg3 = deg2.reshape(NC, NPAD, 1)
    return pl.pallas_call(
        _tc_scale_body,
        grid=(grid,),
        in_specs=[
            pl.BlockSpec((R, DI), lambda i: (i, 0)),
            pl.BlockSpec((NC, R, 1), lambda i: (0, i, 0)),
        ],
        out_specs=pl.BlockSpec((R, DI), lambda i: (i, 0)),
        out_shape=jax.ShapeDtypeStruct((N, DI), jnp.float32),
    )(x, deg3)


def _sc_agg_body(src_hbm, dst_hbm, y_hbm, zrows_hbm, z_out,
                 sidx, didx, rows0, rows1, zsh, isem, g0, g1):
    c = lax.axis_index("c")
    s = lax.axis_index("s")
    wid = s * NC + c
    rows = (rows0, rows1)
    gsems = (g0, g1)
    pltpu.sync_copy(zrows_hbm.at[pl.ds(s * RPTZ, RPTZ)],
                    zsh.at[pl.ds(s * RPTZ, RPTZ)])

    def stage_idx(p):
        pltpu.async_copy(src_hbm.at[wid, p], sidx, isem)
        pltpu.async_copy(dst_hbm.at[wid, p], didx, isem)

    def wait_idx(p):
        pltpu.make_async_copy(src_hbm.at[wid, p], sidx, isem).wait()
        pltpu.make_async_copy(dst_hbm.at[wid, p], didx, isem).wait()

    def scatter(local, b):
        pltpu.sync_copy(rows[b], zsh.at[didx.at[local]], add=True)

    stage_idx(0)
    wait_idx(0)
    plsc.subcore_barrier()

    def fire(local, b):
        pltpu.async_copy(y_hbm.at[sidx.at[local]], rows[b], gsems[b])

    def wait_gather(local, b):
        pltpu.make_async_copy(y_hbm.at[sidx.at[local]], rows[b],
                              gsems[b]).wait()

    for p in range(NPH):
        if p > 0:
            wait_idx(p)
        fire(0, 0)
        fire(1, 1)

        def pair(i, carry):
            for b in range(NBUF):
                local = 2 * i + b
                wait_gather(local, b)
                scatter(local, b)
                fire(local + 2, b)
            return carry

        lax.fori_loop(0, PHC // 2 - 1, pair, 0)
        for b in range(NBUF):
            wait_gather(PHC - 2 + b, b)
            scatter(PHC - 2 + b, b)
        if p + 1 < NPH:
            stage_idx(p + 1)

    plsc.subcore_barrier()
    pltpu.sync_copy(zsh.at[pl.ds(s * RPTZ, RPTZ)],
                    z_out.at[c, pl.ds(s * RPTZ, RPTZ)])


@functools.cache
def _sc_agg():
    return pl.kernel(
        _sc_agg_body,
        out_type=jax.ShapeDtypeStruct((NC, NPZ, DI), jnp.float32),
        mesh=_mesh(),
        scratch_types=[
            pltpu.VMEM((PHC, CH), jnp.int32),
            pltpu.VMEM((PHC, CH), jnp.int32),
            pltpu.VMEM((CH, DI), jnp.float32),
            pltpu.VMEM((CH, DI), jnp.float32),
            pltpu.VMEM_SHARED((NPZ, DI), jnp.float32),
            pltpu.SemaphoreType.DMA,
            pltpu.SemaphoreType.DMA,
            pltpu.SemaphoreType.DMA,
        ],
    )


def _bspline_bases(v):
    u = (v - T0) * (1.0 / H)
    m = jnp.floor(u)
    f = u - m
    mi = m.astype(jnp.int32)
    f2 = f * f
    f3 = f2 * f
    one_sixth = 1.0 / 6.0
    p0 = f3 * one_sixth
    p1 = (-3.0 * f3 + 3.0 * f2 + 3.0 * f + 1.0) * one_sixth
    p2 = (3.0 * f3 - 6.0 * f2 + 4.0) * one_sixth
    g = 1.0 - f
    p3 = g * g * g * one_sixth
    zero = jnp.zeros_like(v)
    bases = []
    for j in range(NB):
        b = jnp.where(mi == j, p0, zero)
        b = jnp.where(mi == j + 1, p1, b)
        b = jnp.where(mi == j + 2, p2, b)
        b = jnp.where(mi == j + 3, p3, b)
        bases.append(b)
    return bases


def _kan(v, wb_ref, ws_ref):
    base = jnp.dot(v * jax.nn.sigmoid(v), wb_ref[...],
                   preferred_element_type=jnp.float32)
    acc = base
    for j, bj in enumerate(_bspline_bases(v)):
        acc = acc + jnp.dot(bj, ws_ref[j], preferred_element_type=jnp.float32)
    return acc


def _tc_kan_body(x_ref, z_ref, deg_ref, wbl_ref, wsl_ref, wbc_ref, wsc_ref, out_ref):
    xb = x_ref[...]
    zb = z_ref[0] + z_ref[1]
    deg = deg_ref[0] + deg_ref[1]
    dinv = lax.rsqrt(jnp.maximum(deg, 1.0))
    agg = zb * dinv
    out_ref[...] = _kan(xb, wbl_ref, wsl_ref) + _kan(agg, wbc_ref, wsc_ref)


def _tc_kan(x, zfull, deg2, wbl_t, wsl_t, wbc_t, wsc_t):
    R = 400
    grid = N // R
    deg3 = deg2.reshape(NC, NPAD, 1)
    wspec = pl.BlockSpec((DI, DO), lambda i: (0, 0))
    sspec = pl.BlockSpec((NB, DI, DO), lambda i: (0, 0, 0))
    return pl.pallas_call(
        _tc_kan_body,
        grid=(grid,),
        in_specs=[
            pl.BlockSpec((R, DI), lambda i: (i, 0)),
            pl.BlockSpec((NC, R, DI), lambda i: (0, i, 0)),
            pl.BlockSpec((NC, R, 1), lambda i: (0, i, 0)),
            wspec, sspec, wspec, sspec,
        ],
        out_specs=pl.BlockSpec((R, DO), lambda i: (i, 0)),
        out_shape=jax.ShapeDtypeStruct((N, DO), jnp.float32),
    )(x, zfull, deg3, wbl_t, wsl_t, wbc_t, wsc_t)


def kernel(x, base_w_l, spline_w_l, base_w_c, spline_w_c, edge_index):
    loop = jnp.arange(N, dtype=jnp.int32)
    pad_src = jnp.arange(E_PAD - E_TOT, dtype=jnp.int32) % N
    pad_dst = N + jnp.arange(E_PAD - E_TOT, dtype=jnp.int32) % (NPZ - N)
    src_flat = jnp.concatenate([edge_index[0], loop, pad_src])
    dst_flat = jnp.concatenate([edge_index[1], loop, pad_dst])
    src4 = src_flat.reshape(NW, NPH, PHC, CH)
    dst4 = dst_flat.reshape(NW, NPH, PHC, CH)
    dst3 = dst_flat.reshape(NW, NCHUNK, CH)
    zdeg = jnp.zeros((NPAD,), jnp.float32)
    zrows = jnp.zeros((NPZ, DI), jnp.float32)
    wbl_t = base_w_l.T
    wbc_t = base_w_c.T
    wsl_t = jnp.transpose(spline_w_l, (2, 1, 0))
    wsc_t = jnp.transpose(spline_w_c, (2, 1, 0))

    deg2 = _sc_hist()(dst3, zdeg)
    y = _tc_scale(x, deg2)
    zfull = _sc_agg()(src4, dst4, y, zrows)
    return _tc_kan(x, zfull, deg2, wbl_t, wsl_t, wbc_t, wsc_t)


# speedup vs baseline: 4.0611x; 1.0621x over previous
import functools

import jax
import jax.numpy as jnp
from jax import lax
from jax.experimental import pallas as pl
from jax.experimental.pallas import tpu as pltpu
from jax.experimental.pallas import tpu_sc as plsc

N = 10000
DI = 128
DO = 128
E_RAW = 320000
E_TOT = E_RAW + N
NC = 2
NS = 16
NW = NC * NS
CH = 128
NBUF = 2
NPH = 2
PHC = 42
NCHUNK = NPH * PHC
E_PAD = NW * NCHUNK * CH
NPAD = 10240
RPT = NPAD // NS
NPZ = 10240
RPTZ = NPZ // NS

GRID_SIZE = 5
SPLINE_ORDER = 3
NB = GRID_SIZE + SPLINE_ORDER
H = 2.0 / GRID_SIZE
T0 = -1.0 - SPLINE_ORDER * H

@functools.cache
def _mesh():
    return plsc.VectorSubcoreMesh(
        core_axis_name="c", subcore_axis_name="s", num_cores=NC, num_subcores=NS)


def _sc_hist_body(dst_hbm, zdeg_hbm, deg_out, didx, ones, degsh, sem):
    c = lax.axis_index("c")
    s = lax.axis_index("s")
    wid = s * NC + c
    for i in range(CH // 16):
        ones[pl.ds(i * 16, 16)] = jnp.ones((16,), jnp.float32)
    pltpu.sync_copy(zdeg_hbm.at[pl.ds(s * RPT, RPT)], degsh.at[pl.ds(s * RPT, RPT)])
    pltpu.async_copy(dst_hbm.at[wid], didx, sem).wait()
    plsc.subcore_barrier()

    def step(j, carry):
        pltpu.sync_copy(ones, degsh.at[didx.at[j]], add=True)
        return carry

    lax.fori_loop(0, NCHUNK, step, 0)
    plsc.subcore_barrier()
    pltpu.sync_copy(degsh.at[pl.ds(s * RPT, RPT)], deg_out.at[c, pl.ds(s * RPT, RPT)])


@functools.cache
def _sc_hist():
    return pl.kernel(
        _sc_hist_body,
        out_type=jax.ShapeDtypeStruct((NC, NPAD), jnp.float32),
        mesh=_mesh(),
        scratch_types=[
            pltpu.VMEM((NCHUNK, CH), jnp.int32),
            pltpu.VMEM((CH,), jnp.float32),
            pltpu.VMEM_SHARED((NPAD,), jnp.float32),
            pltpu.SemaphoreType.DMA,
        ],
    )


def _tc_scale_body(x_ref, deg_ref, y_ref):
    deg = deg_ref[0] + deg_ref[1]
    dinv = lax.rsqrt(jnp.maximum(deg, 1.0))
    y_ref[...] = x_ref[...] * dinv


def _tc_scale(x, deg2):
    R = 400
    grid = N // R
    deg3 = deg2.reshape(NC, NPAD, 1)
    return pl.pallas_call(
        _tc_scale_body,
        grid=(grid,),
        in_specs=[
            pl.BlockSpec((R, DI), lambda i: (i, 0)),
            pl.BlockSpec((NC, R, 1), lambda i: (0, i, 0)),
        ],
        out_specs=pl.BlockSpec((R, DI), lambda i: (i, 0)),
        out_shape=jax.ShapeDtypeStruct((N, DI), jnp.float32),
    )(x, deg3)


def _sc_agg_body(src_hbm, dst_hbm, y_hbm, z_out,
                 sidx, didx, rows0, rows1, zsh, isem, g0, g1):
    c = lax.axis_index("c")
    s = lax.axis_index("s")
    wid = s * NC + c
    rows = (rows0, rows1)
    gsems = (g0, g1)

    def zrow(i, carry):
        for j in range(DI // 16):
            rows0[i, pl.ds(j * 16, 16)] = jnp.zeros((16,), jnp.float32)
        return carry

    lax.fori_loop(0, CH, zrow, 0)
    for k in range(RPTZ // CH):
        pltpu.sync_copy(rows0, zsh.at[pl.ds(s * RPTZ + k * CH, CH)])

    def stage_idx(p):
        pltpu.async_copy(src_hbm.at[wid, p], sidx, isem)
        pltpu.async_copy(dst_hbm.at[wid, p], didx, isem)

    def wait_idx(p):
        pltpu.make_async_copy(src_hbm.at[wid, p], sidx, isem).wait()
        pltpu.make_async_copy(dst_hbm.at[wid, p], didx, isem).wait()

    def scatter(local, b):
        pltpu.sync_copy(rows[b], zsh.at[didx.at[local]], add=True)

    stage_idx(0)
    wait_idx(0)
    plsc.subcore_barrier()

    def fire(local, b):
        pltpu.async_copy(y_hbm.at[sidx.at[local]], rows[b], gsems[b])

    def wait_gather(local, b):
        pltpu.make_async_copy(y_hbm.at[sidx.at[local]], rows[b],
                              gsems[b]).wait()

    for p in range(NPH):
        if p > 0:
            wait_idx(p)
        fire(0, 0)
        fire(1, 1)

        def pair(i, carry):
            for b in range(NBUF):
                local = 2 * i + b
                wait_gather(local, b)
                scatter(local, b)
                fire(local + 2, b)
            return carry

        lax.fori_loop(0, PHC // 2 - 1, pair, 0)
        for b in range(NBUF):
            wait_gather(PHC - 2 + b, b)
            scatter(PHC - 2 + b, b)
        if p + 1 < NPH:
            stage_idx(p + 1)

    plsc.subcore_barrier()
    pltpu.sync_copy(zsh.at[pl.ds(s * RPTZ, RPTZ)],
                    z_out.at[c, pl.ds(s * RPTZ, RPTZ)])


@functools.cache
def _sc_agg():
    return pl.kernel(
        _sc_agg_body,
        out_type=jax.ShapeDtypeStruct((NC, NPZ, DI), jnp.float32),
        mesh=_mesh(),
        scratch_types=[
            pltpu.VMEM((PHC, CH), jnp.int32),
            pltpu.VMEM((PHC, CH), jnp.int32),
            pltpu.VMEM((CH, DI), jnp.float32),
            pltpu.VMEM((CH, DI), jnp.float32),
            pltpu.VMEM_SHARED((NPZ, DI), jnp.float32),
            pltpu.SemaphoreType.DMA,
            pltpu.SemaphoreType.DMA,
            pltpu.SemaphoreType.DMA,
        ],
    )


def _bspline_bases(v):
    u = (v - T0) * (1.0 / H)
    m = jnp.floor(u)
    f = u - m
    mi = m.astype(jnp.int32)
    f2 = f * f
    f3 = f2 * f
    one_sixth = 1.0 / 6.0
    p0 = f3 * one_sixth
    p1 = (-3.0 * f3 + 3.0 * f2 + 3.0 * f + 1.0) * one_sixth
    p2 = (3.0 * f3 - 6.0 * f2 + 4.0) * one_sixth
    g = 1.0 - f
    p3 = g * g * g * one_sixth
    zero = jnp.zeros_like(v)
    bases = []
    for j in range(NB):
        b = jnp.where(mi == j, p0, zero)
        b = jnp.where(mi == j + 1, p1, b)
        b = jnp.where(mi == j + 2, p2, b)
        b = jnp.where(mi == j + 3, p3, b)
        bases.append(b)
    return bases


def _kan(v, wb_ref, ws_ref):
    base = jnp.dot(v * jax.nn.sigmoid(v), wb_ref[...],
                   preferred_element_type=jnp.float32)
    acc = base
    for j, bj in enumerate(_bspline_bases(v)):
        acc = acc + jnp.dot(bj, ws_ref[j], preferred_element_type=jnp.float32)
    return acc


def _tc_kan_layer_body(x_ref, wbl_ref, wsl_ref, out_ref):
    out_ref[...] = _kan(x_ref[...], wbl_ref, wsl_ref)


def _tc_kan_layer(x, wbl_t, wsl_t):
    R = 400
    return pl.pallas_call(
        _tc_kan_layer_body,
        grid=(N // R,),
        in_specs=[
            pl.BlockSpec((R, DI), lambda i: (i, 0)),
            pl.BlockSpec((DI, DO), lambda i: (0, 0)),
            pl.BlockSpec((NB, DI, DO), lambda i: (0, 0, 0)),
        ],
        out_specs=pl.BlockSpec((R, DO), lambda i: (i, 0)),
        out_shape=jax.ShapeDtypeStruct((N, DO), jnp.float32),
    )(x, wbl_t, wsl_t)


def _tc_kan_conv_body(z_ref, deg_ref, hl_ref, wbc_ref, wsc_ref, out_ref):
    zb = z_ref[0] + z_ref[1]
    deg = deg_ref[0] + deg_ref[1]
    dinv = lax.rsqrt(jnp.maximum(deg, 1.0))
    agg = zb * dinv
    out_ref[...] = hl_ref[...] + _kan(agg, wbc_ref, wsc_ref)


def _tc_kan_conv(zfull, deg2, hl, wbc_t, wsc_t):
    R = 400
    deg3 = deg2.reshape(NC, NPAD, 1)
    return pl.pallas_call(
        _tc_kan_conv_body,
        grid=(N // R,),
        in_specs=[
            pl.BlockSpec((NC, R, DI), lambda i: (0, i, 0)),
            pl.BlockSpec((NC, R, 1), lambda i: (0, i, 0)),
            pl.BlockSpec((R, DO), lambda i: (i, 0)),
            pl.BlockSpec((DI, DO), lambda i: (0, 0)),
            pl.BlockSpec((NB, DI, DO), lambda i: (0, 0, 0)),
        ],
        out_specs=pl.BlockSpec((R, DO), lambda i: (i, 0)),
        out_shape=jax.ShapeDtypeStruct((N, DO), jnp.float32),
    )(zfull, deg3, hl, wbc_t, wsc_t)


def kernel(x, base_w_l, spline_w_l, base_w_c, spline_w_c, edge_index):
    loop = jnp.arange(N, dtype=jnp.int32)
    pad_src = jnp.arange(E_PAD - E_TOT, dtype=jnp.int32) % N
    pad_dst = N + jnp.arange(E_PAD - E_TOT, dtype=jnp.int32) % (NPZ - N)
    src_flat = jnp.concatenate([edge_index[0], loop, pad_src])
    dst_flat = jnp.concatenate([edge_index[1], loop, pad_dst])
    src4 = src_flat.reshape(NW, NPH, PHC, CH)
    dst4 = dst_flat.reshape(NW, NPH, PHC, CH)
    dst3 = dst_flat.reshape(NW, NCHUNK, CH)
    zdeg = jnp.zeros((NPAD,), jnp.float32)
    wbl_t = base_w_l.T
    wbc_t = base_w_c.T
    wsl_t = jnp.transpose(spline_w_l, (2, 1, 0))
    wsc_t = jnp.transpose(spline_w_c, (2, 1, 0))

    deg2 = _sc_hist()(dst3, zdeg)
    y = _tc_scale(x, deg2)
    zfull = _sc_agg()(src4, dst4, y)
    hl = _tc_kan_layer(x, wbl_t, wsl_t)
    return _tc_kan_conv(zfull, deg2, hl, wbc_t, wsc_t)
